# baseline (speedup 1.0000x reference)
import functools

import jax
import jax.numpy as jnp
from jax import lax
from jax.experimental import pallas as pl
from jax.experimental.pallas import tpu as pltpu
from jax.experimental.pallas import tpu_sc as plsc

N = 10000
E = 320000
D = 128
NC = 2
NS = 16
NW = NC * NS
L = 16
CH = 128
NCH = 80
G = 8
NG = NCH // G
EPW = NCH * CH
E_PAD = NW * EPW
N_PAD = 10240
RPT = N_PAD // NS
SUB = (128, 128, 128, 128, 128)

_MESH = dict(core_axis_name="c", subcore_axis_name="s", num_cores=NC,
             num_subcores=NS)


def _zero_rows(rows_v):
    def zr(i, _):
        for q in range(D // L):
            rows_v[i, pl.ds(q * L, L)] = jnp.zeros((L,), jnp.float32)
        return 0
    lax.fori_loop(0, CH, zr, 0)


def _rsqrt16(d):
    d = jnp.maximum(d, jnp.full((L,), 1.0, jnp.float32))
    i = plsc.bitcast(d, jnp.int32)
    i = (jnp.full((L,), 0x5F3759DF, jnp.int32)
         - lax.shift_right_logical(i, jnp.full((L,), 1, jnp.int32)))
    y = plsc.bitcast(i, jnp.float32)
    half = jnp.full((L,), 0.5, jnp.float32)
    th = jnp.full((L,), 1.5, jnp.float32)
    for _ in range(3):
        y = y * (th - half * d * y * y)
    return y


def _sc_deg_agg_body(x_hbm, src_hbm, dst_hbm, cw_hbm,
                     agg_out, ns_out, nd_out,
                     src_g, dst_g, c_g, rows_a, rows_b, ones_v, nd_v, norm_v,
                     od_sp, id_sp, ns_sp, agg_sp,
                     ga_sem, gb_sem, sa_sem, sb_sem):
    cid = lax.axis_index("c")
    sid = lax.axis_index("s")
    wid = sid * NC + cid
    base = sid * RPT
    for t in range(CH // L):
        ones_v[pl.ds(t * L, L)] = jnp.ones((L,), jnp.float32)
    _zero_rows(rows_a)
    for p in range(RPT // CH):
        pltpu.sync_copy(rows_a.at[0], od_sp.at[pl.ds(base + p * CH, CH)])
        pltpu.sync_copy(rows_a.at[0], id_sp.at[pl.ds(base + p * CH, CH)])
    plsc.subcore_barrier()
    for co in range(NC):
        w2 = sid * NC + co

        def dgrp(grp, _, w2=w2):
            pltpu.sync_copy(src_hbm.at[w2, pl.ds(grp * G, G)], src_g)
            pltpu.sync_copy(dst_hbm.at[w2, pl.ds(grp * G, G)], dst_g)
            for j in range(G):
                pltpu.sync_copy(ones_v, od_sp.at[src_g.at[j]], add=True)
                pltpu.sync_copy(ones_v, id_sp.at[dst_g.at[j]], add=True)
            return 0
        lax.fori_loop(0, NG, dgrp, 0)
    plsc.subcore_barrier()
    pltpu.sync_copy(id_sp.at[pl.ds(base, RPT)], nd_v)
    for t in range(RPT // L):
        nd_v[pl.ds(t * L, L)] = _rsqrt16(nd_v[pl.ds(t * L, L)])
    pltpu.sync_copy(nd_v, nd_out.at[cid, pl.ds(base, RPT)])
    pltpu.sync_copy(od_sp.at[pl.ds(base, RPT)], norm_v.at[pl.ds(0, RPT)])
    for t in range(RPT // L):
        norm_v[pl.ds(t * L, L)] = _rsqrt16(norm_v[pl.ds(t * L, L)])
    pltpu.sync_copy(norm_v.at[pl.ds(0, RPT)], ns_sp.at[pl.ds(base, RPT)])
    pltpu.sync_copy(norm_v.at[pl.ds(0, RPT)],
                    ns_out.at[cid, pl.ds(base, RPT)])
    off = 0
    for sz in SUB:
        pltpu.sync_copy(rows_a.at[pl.ds(0, sz)],
                        agg_sp.at[pl.ds(base + off, sz), :])
        off += sz
    plsc.subcore_barrier()
    pltpu.sync_copy(ns_sp, norm_v)

    def group(grp, _):
        pltpu.sync_copy(src_hbm.at[wid, pl.ds(grp * G, G)], src_g)
        pltpu.sync_copy(dst_hbm.at[wid, pl.ds(grp * G, G)], dst_g)
        pltpu.sync_copy(cw_hbm.at[wid, pl.ds(grp * G, G)], c_g)

        def cbody(j2, _):
            for t in range(CH // L):
                idx = src_g[j2, pl.ds(t * L, L)]
                nv = plsc.load_gather(norm_v, [idx])
                c_g[j2, pl.ds(t * L, L)] = c_g[j2, pl.ds(t * L, L)] * nv
            return 0
        lax.fori_loop(0, G, cbody, 0)
        bufs = ((rows_a, ga_sem, sa_sem), (rows_b, gb_sem, sb_sem))
        pltpu.async_copy(x_hbm.at[src_g.at[0]], rows_a, ga_sem)
        for j in range(G):
            rows, gsem, ssem = bufs[j % 2]
            orows, ogsem, ossem = bufs[(j + 1) % 2]
            pltpu.make_async_copy(x_hbm.at[src_g.at[j]], rows, gsem).wait()
            if j < G - 1:
                if j >= 1:
                    pltpu.make_async_copy(
                        orows, agg_sp.at[dst_g.at[j - 1]], ossem).wait()
                pltpu.async_copy(x_hbm.at[src_g.at[j + 1]], orows, ogsem)

            def scale(t, _, j=j, rows=rows):
                cvec = c_g[j, pl.ds(t * L, L)]
                for k in range(L):
                    cv = lax.broadcast(cvec[k], (L,))
                    row = t * L + k
                    for q in range(D // L):
                        rows[row, pl.ds(q * L, L)] = (
                            rows[row, pl.ds(q * L, L)] * cv)
                return 0
            lax.fori_loop(0, CH // L, scale, 0)
            pltpu.async_copy(rows, agg_sp.at[dst_g.at[j]], ssem, add=True)
        pltpu.make_async_copy(rows_a, agg_sp.at[dst_g.at[G - 2]],
                              sa_sem).wait()
        pltpu.make_async_copy(rows_b, agg_sp.at[dst_g.at[G - 1]],
                              sb_sem).wait()
        return 0
    lax.fori_loop(0, NG, group, 0)
    plsc.subcore_barrier()
    off = 0
    for sz in SUB:
        pltpu.sync_copy(agg_sp.at[pl.ds(base + off, sz), :],
                        rows_a.at[pl.ds(0, sz)])

        def nrow(g, _, off=off):
            nvec = nd_v[pl.ds(off + g * L, L)]
            for k in range(L):
                cv = lax.broadcast(nvec[k], (L,))
                row = g * L + k
                for q in range(D // L):
                    rows_a[row, pl.ds(q * L, L)] = (
                        rows_a[row, pl.ds(q * L, L)] * cv)
            return 0
        lax.fori_loop(0, sz // L, nrow, 0)
        pltpu.sync_copy(rows_a.at[pl.ds(0, sz)],
                        agg_out.at[cid, pl.ds(base + off, sz), :])
        off += sz


def _sc_deg_agg(x, srcw, dstw, eww):
    mesh = plsc.VectorSubcoreMesh(**_MESH)
    f = pl.kernel(
        _sc_deg_agg_body,
        out_type=(jax.ShapeDtypeStruct((NC, N_PAD, D), jnp.float32),
                  jax.ShapeDtypeStruct((NC, N_PAD), jnp.float32),
                  jax.ShapeDtypeStruct((NC, N_PAD), jnp.float32)),
        mesh=mesh,
        compiler_params=pltpu.CompilerParams(needs_layout_passes=False),
        scratch_types=[
            pltpu.VMEM((G, CH), jnp.int32),
            pltpu.VMEM((G, CH), jnp.int32),
            pltpu.VMEM((G, CH), jnp.float32),
            pltpu.VMEM((CH, D), jnp.float32),
            pltpu.VMEM((CH, D), jnp.float32),
            pltpu.VMEM((CH,), jnp.float32),
            pltpu.VMEM((RPT,), jnp.float32),
            pltpu.VMEM((N_PAD,), jnp.float32),
            pltpu.VMEM_SHARED((N_PAD,), jnp.float32),
            pltpu.VMEM_SHARED((N_PAD,), jnp.float32),
            pltpu.VMEM_SHARED((N_PAD,), jnp.float32),
            pltpu.VMEM_SHARED((N_PAD, D), jnp.float32),
            pltpu.SemaphoreType.DMA,
            pltpu.SemaphoreType.DMA,
            pltpu.SemaphoreType.DMA,
            pltpu.SemaphoreType.DMA,
        ],
    )
    return f(x, srcw, dstw, eww)


def _sc_agg_body(x_hbm, src_hbm, dst_hbm, cw_hbm, nd_hbm,
                 agg_out,
                 src_g, dst_g, c_g, rows_a, rows_b, nd_v, agg_sp,
                 ga_sem, gb_sem, sa_sem, sb_sem):
    rows_v = rows_a
    cid = lax.axis_index("c")
    sid = lax.axis_index("s")
    wid = sid * NC + cid
    base = sid * RPT
    pltpu.sync_copy(nd_hbm.at[pl.ds(base, RPT)], nd_v)
    _zero_rows(rows_v)
    off = 0
    for sz in SUB:
        pltpu.sync_copy(rows_v.at[pl.ds(0, sz)],
                        agg_sp.at[pl.ds(base + off, sz), :])
        off += sz
    plsc.subcore_barrier()

    def group(grp, _):
        pltpu.sync_copy(src_hbm.at[wid, pl.ds(grp * G, G)], src_g)
        pltpu.sync_copy(dst_hbm.at[wid, pl.ds(grp * G, G)], dst_g)
        pltpu.sync_copy(cw_hbm.at[wid, pl.ds(grp * G, G)], c_g)
        bufs = ((rows_a, ga_sem, sa_sem), (rows_b, gb_sem, sb_sem))
        pltpu.async_copy(x_hbm.at[src_g.at[0]], rows_a, ga_sem)
        for j in range(G):
            rows, gsem, ssem = bufs[j % 2]
            orows, ogsem, ossem = bufs[(j + 1) % 2]
            pltpu.make_async_copy(x_hbm.at[src_g.at[j]], rows,
                                  gsem).wait()
            if j < G - 1:
                if j >= 1:
                    pltpu.make_async_copy(
                        orows, agg_sp.at[dst_g.at[j - 1]], ossem).wait()
                pltpu.async_copy(x_hbm.at[src_g.at[j + 1]], orows, ogsem)

            def scale(t, _, j=j, rows=rows):
                cvec = c_g[j, pl.ds(t * L, L)]
                for k in range(L):
                    cv = lax.broadcast(cvec[k], (L,))
                    row = t * L + k
                    for q in range(D // L):
                        rows[row, pl.ds(q * L, L)] = (
                            rows[row, pl.ds(q * L, L)] * cv)
                return 0
            lax.fori_loop(0, CH // L, scale, 0)
            pltpu.async_copy(rows, agg_sp.at[dst_g.at[j]], ssem, add=True)
        pltpu.make_async_copy(rows_a, agg_sp.at[dst_g.at[G - 2]],
                              sa_sem).wait()
        pltpu.make_async_copy(rows_b, agg_sp.at[dst_g.at[G - 1]],
                              sb_sem).wait()
        return 0
    lax.fori_loop(0, NG, group, 0)
    plsc.subcore_barrier()
    off = 0
    for sz in SUB:
        pltpu.sync_copy(agg_sp.at[pl.ds(base + off, sz), :],
                        rows_v.at[pl.ds(0, sz)])

        def nrow(g, _, off=off):
            nvec = nd_v[pl.ds(off + g * L, L)]
            for k in range(L):
                cv = lax.broadcast(nvec[k], (L,))
                row = g * L + k
                for q in range(D // L):
                    rows_v[row, pl.ds(q * L, L)] = (
                        rows_v[row, pl.ds(q * L, L)] * cv)
            return 0
        lax.fori_loop(0, sz // L, nrow, 0)
        pltpu.sync_copy(rows_v.at[pl.ds(0, sz)],
                        agg_out.at[cid, pl.ds(base + off, sz), :])
        off += sz


def _sc_agg(x, srcw, dstw, eww, nd):
    mesh = plsc.VectorSubcoreMesh(**_MESH)
    scratch = [
        pltpu.VMEM((G, CH), jnp.int32),
        pltpu.VMEM((G, CH), jnp.int32),
        pltpu.VMEM((G, CH), jnp.float32),
        pltpu.VMEM((CH, D), jnp.float32),
        pltpu.VMEM((CH, D), jnp.float32),
        pltpu.VMEM((RPT,), jnp.float32),
        pltpu.VMEM_SHARED((N_PAD, D), jnp.float32),
        pltpu.SemaphoreType.DMA,
        pltpu.SemaphoreType.DMA,
        pltpu.SemaphoreType.DMA,
        pltpu.SemaphoreType.DMA,
    ]
    f = pl.kernel(
        _sc_agg_body,
        out_type=(jax.ShapeDtypeStruct((NC, N_PAD, D), jnp.float32),),
        mesh=mesh,
        compiler_params=pltpu.CompilerParams(needs_layout_passes=False),
        scratch_types=scratch,
    )
    (agg,) = f(x, srcw, dstw, eww, nd)
    return agg


def _tc_ew_body(e_ref, l1w_ref, l1b_ref, l2w_ref, l2b_ref, o_ref):
    w_eff = jnp.sum(l1w_ref[...] * l2w_ref[...][None, :], axis=1)
    b_eff = jnp.sum(l1b_ref[...] * l2w_ref[...]) + l2b_ref[...][0]
    o_ref[...] = jnp.sum(e_ref[...] * w_eff[None, :], axis=1) + b_eff


def _tc_ew(edges_pad, l1w, l1b, l2w_flat, l2b):
    blk = 2048
    grid = E_PAD // blk
    return pl.pallas_call(
        _tc_ew_body,
        grid=(grid,),
        in_specs=[
            pl.BlockSpec((blk, 16), lambda i: (i, 0)),
            pl.BlockSpec((16, 8), lambda i: (0, 0)),
            pl.BlockSpec((8,), lambda i: (0,)),
            pl.BlockSpec((8,), lambda i: (0,)),
            pl.BlockSpec((1,), lambda i: (0,)),
        ],
        out_specs=pl.BlockSpec((blk,), lambda i: (i,)),
        out_shape=jax.ShapeDtypeStruct((E_PAD,), jnp.float32),
    )(edges_pad, l1w, l1b, l2w_flat, l2b)


def _make_tc_layer_body(scale_out):
    def body(aggp_ref, w_ref, b_ref, ns_ref, o_ref):
        a = aggp_ref[0] + aggp_ref[1]
        h = jnp.dot(a, w_ref[...], preferred_element_type=jnp.float32,
                    precision=lax.Precision.HIGHEST)
        h = jnp.maximum(h + b_ref[...][None, :], 0.0)
        if scale_out:
            i = pl.program_id(0)
            nsb = ns_ref[pl.ds(i * RPT, RPT)]
            h = h * nsb[:, None]
        o_ref[...] = h
    return body


def _tc_layer(aggp, w, b, ns, scale_out):
    rb = RPT
    grid = N_PAD // rb
    return pl.pallas_call(
        _make_tc_layer_body(scale_out),
        grid=(grid,),
        in_specs=[
            pl.BlockSpec((NC, rb, D), lambda i: (0, i, 0)),
            pl.BlockSpec((D, D), lambda i: (0, 0)),
            pl.BlockSpec((D,), lambda i: (0,)),
            pl.BlockSpec((N_PAD,), lambda i: (0,)),
        ],
        out_specs=pl.BlockSpec((rb, D), lambda i: (i, 0)),
        out_shape=jax.ShapeDtypeStruct((N_PAD, D), jnp.float32),
    )(aggp, w, b, ns)


def kernel(inputs, edge_index, edges, W1, b1, W2, b2, lin1_W, lin1_b,
           lin2_W, lin2_b):
    pad = E_PAD - E
    trash = jnp.full((pad,), N, jnp.int32)
    srcw = jnp.concatenate([edge_index[0], trash]).reshape(NW, NCH, CH)
    dstw = jnp.concatenate([edge_index[1], trash]).reshape(NW, NCH, CH)
    x_pad = jnp.pad(inputs, ((0, N_PAD - N), (0, 0)))
    edges_pad = jnp.pad(edges, ((0, pad), (0, 0)))
    l2w_flat = lin2_W.reshape(8)

    ew = _tc_ew(edges_pad, lin1_W, lin1_b, l2w_flat, lin2_b)
    eww = ew.reshape(NW, NCH, CH)
    agg1, nso, ndo = _sc_deg_agg(x_pad, srcw, dstw, eww)
    ns = nso[0]
    nd = ndo[0]
    h_s = _tc_layer(agg1, W1, b1, ns, True)
    agg2 = _sc_agg(h_s, srcw, dstw, eww, nd)
    out_full = _tc_layer(agg2, W2, b2, ns, False)
    return out_full[:N]
